# baseline (speedup 1.0000x reference)
import functools

import jax
import jax.numpy as jnp
import numpy as np
from jax import lax
from jax.experimental import pallas as pl
from jax.experimental.pallas import tpu as pltpu
from jax.experimental.pallas import tpu_sc as plsc

_N = 10000
_E = 320000
_D = 128
_DE = 16
_G = 50

_NC = 2
_NS = 16
_NW = _NC * _NS
_EPT = _E // _NW
_C = 40
_NCHUNK = _EPT // _C
_ROWS = 624
_TAIL = _N - _NS * _ROWS


def _dense_pre(h, w1i_t, w1j_t):
    B = 1000

    def body(h_ref, wi_ref, wj_ref, pi_ref, pj_ref):
        hb = h_ref[...]
        pi_ref[...] = jnp.dot(hb, wi_ref[...], preferred_element_type=jnp.float32)
        pj_ref[...] = jnp.dot(hb, wj_ref[...], preferred_element_type=jnp.float32)

    return pl.pallas_call(
        body,
        grid=(_N // B,),
        in_specs=[
            pl.BlockSpec((B, _D), lambda i: (i, 0)),
            pl.BlockSpec((_D, _D), lambda i: (0, 0)),
            pl.BlockSpec((_D, _D), lambda i: (0, 0)),
        ],
        out_specs=[pl.BlockSpec((B, _D), lambda i: (i, 0))] * 2,
        out_shape=[jax.ShapeDtypeStruct((_N, _D), jnp.float32)] * 2,
    )(h, w1i_t, w1j_t)


def _dense_q(edge_attr, w1e_t, b1):
    B = 1000
    EP = _E // 8

    def body(e_ref, w_ref, b_ref, q_ref):
        q_ref[...] = (
            jnp.dot(e_ref[...], w_ref[...], preferred_element_type=jnp.float32)
            + b_ref[...]
        )

    ea8 = edge_attr.reshape(EP, 8 * _DE)
    wk = jnp.kron(jnp.eye(8, dtype=jnp.float32), w1e_t)
    bt = jnp.tile(b1.reshape(-1), 8).reshape(1, 8 * _D)
    out = pl.pallas_call(
        body,
        grid=(EP // B,),
        in_specs=[
            pl.BlockSpec((B, 8 * _DE), lambda i: (i, 0)),
            pl.BlockSpec((8 * _DE, 8 * _D), lambda i: (0, 0)),
            pl.BlockSpec((1, 8 * _D), lambda i: (0, 0)),
        ],
        out_specs=pl.BlockSpec((B, 8 * _D), lambda i: (i, 0)),
        out_shape=jax.ShapeDtypeStruct((EP, 8 * _D), jnp.float32),
    )(ea8, wk, bt)
    return out.reshape(_E, _D)


def _lstm_block(refs_h, s0, s1, d0, d1, w2, b2, wih, whh, bs, c=None):
    a = jnp.dot(s0 + s1, w2, preferred_element_type=jnp.float32) + (d0 + d1) * b2
    gates = (
        jnp.dot(refs_h, wih, preferred_element_type=jnp.float32)
        + jnp.dot(a, whh, preferred_element_type=jnp.float32)
        + bs
    )
    ig = jax.nn.sigmoid(gates[:, :_D])
    fg = jax.nn.sigmoid(gates[:, _D:2 * _D])
    gg = jnp.tanh(gates[:, 2 * _D:3 * _D])
    og = jax.nn.sigmoid(gates[:, 3 * _D:])
    cn = ig * gg if c is None else fg * c + ig * gg
    return og * jnp.tanh(cn), cn


def _post_pre(s_parts, deg_parts, h, w2_t, b2, wih_t, whh_t, bsum,
              w1i_t, w1j_t):
    B = 1000

    def body(s0_ref, s1_ref, d0_ref, d1_ref, h_ref,
             w2_ref, b2_ref, wih_ref, whh_ref, bs_ref, wi_ref, wj_ref,
             h2_ref, c2_ref, pi_ref, pj_ref):
        h2, cn = _lstm_block(h_ref[...], s0_ref[...], s1_ref[...],
                             d0_ref[...], d1_ref[...], w2_ref[...],
                             b2_ref[...], wih_ref[...], whh_ref[...],
                             bs_ref[...])
        h2_ref[...] = h2
        c2_ref[...] = cn
        pi_ref[...] = jnp.dot(h2, wi_ref[...], preferred_element_type=jnp.float32)
        pj_ref[...] = jnp.dot(h2, wj_ref[...], preferred_element_type=jnp.float32)

    row_spec = pl.BlockSpec((B, _D), lambda i: (i, 0))
    col_spec = pl.BlockSpec((B, 1), lambda i: (i, 0))
    full = lambda r, cdim: pl.BlockSpec((r, cdim), lambda i: (0, 0))
    return pl.pallas_call(
        body,
        grid=(_N // B,),
        in_specs=[row_spec, row_spec, col_spec, col_spec, row_spec,
                  full(_D, _D), full(1, _D), full(_D, 4 * _D),
                  full(_D, 4 * _D), full(1, 4 * _D), full(_D, _D),
                  full(_D, _D)],
        out_specs=[row_spec] * 4,
        out_shape=[jax.ShapeDtypeStruct((_N, _D), jnp.float32)] * 4,
    )(s_parts[0], s_parts[1],
      deg_parts[0].reshape(_N, 1), deg_parts[1].reshape(_N, 1), h,
      w2_t, b2, wih_t, whh_t, bsum, w1i_t, w1j_t)


def _post_readout(s_parts, deg_parts, h, c, w2_t, b2, wih_t, whh_t, bsum,
                  gm_t, gm_b, fm_t, fm_b):
    B = 1000

    def body(s0_ref, s1_ref, d0_ref, d1_ref, h_ref, c_ref,
             w2_ref, b2_ref, wih_ref, whh_ref, bs_ref,
             gmw_ref, gmb_ref, fmw_ref, fmb_ref, out_ref):
        i = pl.program_id(0)
        h2, _ = _lstm_block(h_ref[...], s0_ref[...], s1_ref[...],
                            d0_ref[...], d1_ref[...], w2_ref[...],
                            b2_ref[...], wih_ref[...], whh_ref[...],
                            bs_ref[...], c=c_ref[...])
        g = jax.nn.sigmoid(
            jnp.dot(h2, gmw_ref[...], preferred_element_type=jnp.float32)
            + gmb_ref[...])
        hv = (jnp.dot(h2, fmw_ref[...], preferred_element_type=jnp.float32)
              + fmb_ref[...])
        part = jnp.sum(g * hv, axis=0, keepdims=True)

        @pl.when(i == 0)
        def _():
            out_ref[...] = part

        @pl.when(i > 0)
        def _():
            out_ref[...] = out_ref[...] + part

    row_spec = pl.BlockSpec((B, _D), lambda i: (i, 0))
    col_spec = pl.BlockSpec((B, 1), lambda i: (i, 0))
    full = lambda r, cdim: pl.BlockSpec((r, cdim), lambda i: (0, 0))
    out = pl.pallas_call(
        body,
        grid=(_N // B,),
        in_specs=[row_spec, row_spec, col_spec, col_spec, row_spec, row_spec,
                  full(_D, _D), full(1, _D), full(_D, 4 * _D),
                  full(_D, 4 * _D), full(1, 4 * _D),
                  full(_D, _G), full(1, _G), full(_D, _G), full(1, _G)],
        out_specs=pl.BlockSpec((1, _G), lambda i: (0, 0)),
        out_shape=jax.ShapeDtypeStruct((1, _G), jnp.float32),
    )(s_parts[0], s_parts[1],
      deg_parts[0].reshape(_N, 1), deg_parts[1].reshape(_N, 1), h, c,
      w2_t, b2, wih_t, whh_t, bsum,
      gm_t, gm_b.reshape(1, _G), fm_t, fm_b.reshape(1, _G))
    return out.reshape(_G)


def _sc_edge(pi, pj, q, src, dst, z_nd, z_n):
    mesh = plsc.VectorSubcoreMesh(
        core_axis_name="c", subcore_axis_name="s",
        num_cores=_NC, num_subcores=_NS)

    @functools.partial(
        pl.kernel,
        mesh=mesh,
        out_type=(jax.ShapeDtypeStruct((_NC, _N, _D), jnp.float32),
                  jax.ShapeDtypeStruct((_NC, _N), jnp.float32)),
        scratch_types=[
            [pltpu.VMEM((_C,), jnp.int32) for _ in range(4)],
            [pltpu.VMEM((_C,), jnp.int32) for _ in range(4)],
            [pltpu.VMEM((_C, _D), jnp.float32) for _ in range(2)],
            [pltpu.VMEM((_C, _D), jnp.float32) for _ in range(2)],
            [pltpu.VMEM((_C, _D), jnp.float32) for _ in range(2)],
            [pltpu.VMEM((_C, _D), jnp.float32) for _ in range(2)],
            pltpu.VMEM((_C,), jnp.float32),
            pltpu.VMEM_SHARED((_N, _D), jnp.float32),
            pltpu.VMEM_SHARED((_N,), jnp.float32),
            [pltpu.SemaphoreType.DMA for _ in range(4)],
            [pltpu.SemaphoreType.DMA for _ in range(2)],
            [pltpu.SemaphoreType.DMA for _ in range(2)],
        ],
    )
    def k(pi_hbm, pj_hbm, q_hbm, src_hbm, dst_hbm, znd_hbm, zn_hbm,
          s_out, deg_out,
          idx_s, idx_d, buf_i, buf_j, buf_q, buf_r, ones_v, s_sh, deg_sh,
          sem_idx, sem_ld, sem_out):
        cid = lax.axis_index("c")
        sid = lax.axis_index("s")
        wid = cid * _NS + sid

        pltpu.sync_copy(znd_hbm.at[pl.ds(sid * _ROWS, _ROWS)],
                        s_sh.at[pl.ds(sid * _ROWS, _ROWS)])

        @pl.when(sid == _NS - 1)
        def _():
            pltpu.sync_copy(znd_hbm.at[pl.ds(_NS * _ROWS, _TAIL)],
                            s_sh.at[pl.ds(_NS * _ROWS, _TAIL)])

        @pl.when(sid == 0)
        def _():
            pltpu.sync_copy(zn_hbm, deg_sh)

        for t in range(_C // 16):
            ones_v[pl.ds(t * 16, 16)] = jnp.ones((16,), jnp.float32)
        if _C % 16:
            ones_v[pl.ds(_C - 16, 16)] = jnp.ones((16,), jnp.float32)

        plsc.subcore_barrier()

        base = wid * _EPT

        def issue_idx(ch, s):
            off = base + ch * _C
            pltpu.async_copy(src_hbm.at[pl.ds(off, _C)], idx_s[s], sem_idx[s])
            pltpu.async_copy(dst_hbm.at[pl.ds(off, _C)], idx_d[s], sem_idx[s])

        def wait_idx(s):
            pltpu.make_async_copy(src_hbm.at[pl.ds(0, _C)], idx_s[s], sem_idx[s]).wait()
            pltpu.make_async_copy(dst_hbm.at[pl.ds(0, _C)], idx_d[s], sem_idx[s]).wait()

        def issue_ld(ch, s, b):
            off = base + ch * _C
            pltpu.async_copy(pi_hbm.at[idx_d[s]], buf_i[b], sem_ld[b])
            pltpu.async_copy(pj_hbm.at[idx_s[s]], buf_j[b], sem_ld[b])
            pltpu.async_copy(q_hbm.at[pl.ds(off, _C)], buf_q[b], sem_ld[b])

        def wait_ld(b):
            pltpu.make_async_copy(pi_hbm.at[idx_d[0]], buf_i[b], sem_ld[b]).wait()
            pltpu.make_async_copy(pj_hbm.at[idx_s[0]], buf_j[b], sem_ld[b]).wait()
            pltpu.make_async_copy(q_hbm.at[pl.ds(0, _C)], buf_q[b], sem_ld[b]).wait()

        def issue_out(s, b):
            pltpu.async_copy(buf_r[b], s_sh.at[idx_d[s]], sem_out[b], add=True)
            pltpu.async_copy(ones_v, deg_sh.at[idx_d[s]], sem_out[b], add=True)

        def wait_out(b):
            pltpu.make_async_copy(buf_r[b], s_sh.at[idx_d[0]], sem_out[b]).wait()
            pltpu.make_async_copy(ones_v, deg_sh.at[idx_d[0]], sem_out[b]).wait()

        def compute(b):
            bi, bj, bq, br = buf_i[b], buf_j[b], buf_q[b], buf_r[b]

            def row(r, carry2):
                for t in range(_D // 16):
                    sl = pl.ds(t * 16, 16)
                    v = bi[r, sl] + bj[r, sl] + bq[r, sl]
                    br[r, sl] = jnp.maximum(v, 0.0)
                return carry2

            lax.fori_loop(0, _C, row, 0)

        issue_idx(0, 0)
        issue_idx(1, 1)
        wait_idx(0)
        issue_ld(0, 0, 0)
        wait_idx(1)
        issue_ld(1, 1, 1)

        def quad(i, carry):
            for kk in range(4):
                ch = 4 * i + kk
                b = kk % 2
                s = kk % 4
                s2 = (kk + 2) % 4

                @pl.when(ch < _NCHUNK)
                def _():
                    wait_ld(b)

                @pl.when(jnp.logical_and(ch >= 2, ch < _NCHUNK + 2))
                def _():
                    wait_out(b)

                @pl.when(ch + 2 < _NCHUNK)
                def _():
                    issue_idx(ch + 2, s2)

                @pl.when(ch < _NCHUNK)
                def _():
                    compute(b)
                    issue_out(s, b)

                @pl.when(ch + 2 < _NCHUNK)
                def _():
                    wait_idx(s2)
                    issue_ld(ch + 2, s2, b)

            return carry

        lax.fori_loop(0, (_NCHUNK + 3) // 4 + 1, quad, 0)

        plsc.subcore_barrier()

        pltpu.sync_copy(s_sh.at[pl.ds(sid * _ROWS, _ROWS)],
                        s_out.at[cid, pl.ds(sid * _ROWS, _ROWS)])

        @pl.when(sid == _NS - 1)
        def _():
            pltpu.sync_copy(s_sh.at[pl.ds(_NS * _ROWS, _TAIL)],
                            s_out.at[cid, pl.ds(_NS * _ROWS, _TAIL)])

        @pl.when(sid == 0)
        def _():
            pltpu.sync_copy(deg_sh, deg_out.at[cid])

    return k(pi, pj, q, src, dst, z_nd, z_n)


def kernel(x, edge_attr, edge_index, W1_0, b1_0, W2_0, b2_0, Wih_0, Whh_0,
           bih_0, bhh_0, W1_1, b1_1, W2_1, b2_1, Wih_1, Whh_1, bih_1, bhh_1,
           gm_W, gm_b, fm_W, fm_b):
    src = edge_index[0].astype(jnp.int32)
    dst = edge_index[1].astype(jnp.int32)
    z_nd = jnp.zeros((_N, _D), jnp.float32)
    z_n = jnp.zeros((_N,), jnp.float32)

    q0 = _dense_q(edge_attr, W1_0[:, 2 * _D:].T, b1_0.reshape(1, _D))
    q1 = _dense_q(edge_attr, W1_1[:, 2 * _D:].T, b1_1.reshape(1, _D))

    pi0, pj0 = _dense_pre(x, W1_0[:, :_D].T, W1_0[:, _D:2 * _D].T)
    s_parts0, deg_parts0 = _sc_edge(pi0, pj0, q0, src, dst, z_nd, z_n)
    h1, c1, pi1, pj1 = _post_pre(
        s_parts0, deg_parts0, x,
        W2_0.T, b2_0.reshape(1, _D), Wih_0.T, Whh_0.T,
        (bih_0 + bhh_0).reshape(1, 4 * _D),
        W1_1[:, :_D].T, W1_1[:, _D:2 * _D].T)
    s_parts1, deg_parts1 = _sc_edge(pi1, pj1, q1, src, dst, z_nd, z_n)
    return _post_readout(
        s_parts1, deg_parts1, h1, c1,
        W2_1.T, b2_1.reshape(1, _D), Wih_1.T, Whh_1.T,
        (bih_1 + bhh_1).reshape(1, 4 * _D),
        gm_W.T, gm_b, fm_W.T, fm_b)


# speedup vs baseline: 1.1530x; 1.1530x over previous
import functools

import jax
import jax.numpy as jnp
import numpy as np
from jax import lax
from jax.experimental import pallas as pl
from jax.experimental.pallas import tpu as pltpu
from jax.experimental.pallas import tpu_sc as plsc

_N = 10000
_E = 320000
_D = 128
_DE = 16
_G = 50

_NC = 2
_NS = 16
_NW = _NC * _NS
_EPT = _E // _NW
_C = 40
_NCHUNK = _EPT // _C
_ROWS = 624
_TAIL = _N - _NS * _ROWS


def _dense_pre(h, w1i_t, w1j_t):
    B = 1000

    def body(h_ref, wi_ref, wj_ref, pi_ref, pj_ref):
        hb = h_ref[...]
        pi_ref[...] = jnp.dot(hb, wi_ref[...], preferred_element_type=jnp.float32)
        pj_ref[...] = jnp.dot(hb, wj_ref[...], preferred_element_type=jnp.float32)

    return pl.pallas_call(
        body,
        grid=(_N // B,),
        in_specs=[
            pl.BlockSpec((B, _D), lambda i: (i, 0)),
            pl.BlockSpec((_D, _D), lambda i: (0, 0)),
            pl.BlockSpec((_D, _D), lambda i: (0, 0)),
        ],
        out_specs=[pl.BlockSpec((B, _D), lambda i: (i, 0))] * 2,
        out_shape=[jax.ShapeDtypeStruct((_N, _D), jnp.float32)] * 2,
    )(h, w1i_t, w1j_t)


def _dense_q2(edge_attr, w1e_t0, b1_0, w1e_t1, b1_1):
    B = 2000

    def body(e_ref, w0_ref, b0_ref, w1_ref, b1_ref, q0_ref, q1_ref):
        eb = e_ref[...]
        q0_ref[...] = (
            jnp.dot(eb, w0_ref[...], preferred_element_type=jnp.float32)
            + b0_ref[...]
        )
        q1_ref[...] = (
            jnp.dot(eb, w1_ref[...], preferred_element_type=jnp.float32)
            + b1_ref[...]
        )

    return pl.pallas_call(
        body,
        grid=(_E // B,),
        in_specs=[
            pl.BlockSpec((B, _DE), lambda i: (i, 0)),
            pl.BlockSpec((_DE, _D), lambda i: (0, 0)),
            pl.BlockSpec((1, _D), lambda i: (0, 0)),
            pl.BlockSpec((_DE, _D), lambda i: (0, 0)),
            pl.BlockSpec((1, _D), lambda i: (0, 0)),
        ],
        out_specs=[pl.BlockSpec((B, _D), lambda i: (i, 0))] * 2,
        out_shape=[jax.ShapeDtypeStruct((_E, _D), jnp.float32)] * 2,
    )(edge_attr, w1e_t0, b1_0, w1e_t1, b1_1)


def _lstm_block(refs_h, s0, s1, d0, d1, w2, b2, wih, whh, bs, c=None):
    a = jnp.dot(s0 + s1, w2, preferred_element_type=jnp.float32) + (d0 + d1) * b2
    gates = (
        jnp.dot(refs_h, wih, preferred_element_type=jnp.float32)
        + jnp.dot(a, whh, preferred_element_type=jnp.float32)
        + bs
    )
    ig = jax.nn.sigmoid(gates[:, :_D])
    fg = jax.nn.sigmoid(gates[:, _D:2 * _D])
    gg = jnp.tanh(gates[:, 2 * _D:3 * _D])
    og = jax.nn.sigmoid(gates[:, 3 * _D:])
    cn = ig * gg if c is None else fg * c + ig * gg
    return og * jnp.tanh(cn), cn


def _post_pre(s_parts, deg_parts, h, w2_t, b2, wih_t, whh_t, bsum,
              w1i_t, w1j_t):
    B = 1000

    def body(s0_ref, s1_ref, d0_ref, d1_ref, h_ref,
             w2_ref, b2_ref, wih_ref, whh_ref, bs_ref, wi_ref, wj_ref,
             h2_ref, c2_ref, pi_ref, pj_ref):
        h2, cn = _lstm_block(h_ref[...], s0_ref[...], s1_ref[...],
                             d0_ref[...], d1_ref[...], w2_ref[...],
                             b2_ref[...], wih_ref[...], whh_ref[...],
                             bs_ref[...])
        h2_ref[...] = h2
        c2_ref[...] = cn
        pi_ref[...] = jnp.dot(h2, wi_ref[...], preferred_element_type=jnp.float32)
        pj_ref[...] = jnp.dot(h2, wj_ref[...], preferred_element_type=jnp.float32)

    row_spec = pl.BlockSpec((B, _D), lambda i: (i, 0))
    col_spec = pl.BlockSpec((B, 1), lambda i: (i, 0))
    full = lambda r, cdim: pl.BlockSpec((r, cdim), lambda i: (0, 0))
    return pl.pallas_call(
        body,
        grid=(_N // B,),
        in_specs=[row_spec, row_spec, col_spec, col_spec, row_spec,
                  full(_D, _D), full(1, _D), full(_D, 4 * _D),
                  full(_D, 4 * _D), full(1, 4 * _D), full(_D, _D),
                  full(_D, _D)],
        out_specs=[row_spec] * 4,
        out_shape=[jax.ShapeDtypeStruct((_N, _D), jnp.float32)] * 4,
    )(s_parts[0], s_parts[1],
      deg_parts[0].reshape(_N, 1), deg_parts[1].reshape(_N, 1), h,
      w2_t, b2, wih_t, whh_t, bsum, w1i_t, w1j_t)


def _post_readout(s_parts, deg_parts, h, c, w2_t, b2, wih_t, whh_t, bsum,
                  gm_t, gm_b, fm_t, fm_b):
    B = 1000

    def body(s0_ref, s1_ref, d0_ref, d1_ref, h_ref, c_ref,
             w2_ref, b2_ref, wih_ref, whh_ref, bs_ref,
             gmw_ref, gmb_ref, fmw_ref, fmb_ref, out_ref):
        i = pl.program_id(0)
        h2, _ = _lstm_block(h_ref[...], s0_ref[...], s1_ref[...],
                            d0_ref[...], d1_ref[...], w2_ref[...],
                            b2_ref[...], wih_ref[...], whh_ref[...],
                            bs_ref[...], c=c_ref[...])
        g = jax.nn.sigmoid(
            jnp.dot(h2, gmw_ref[...], preferred_element_type=jnp.float32)
            + gmb_ref[...])
        hv = (jnp.dot(h2, fmw_ref[...], preferred_element_type=jnp.float32)
              + fmb_ref[...])
        part = jnp.sum(g * hv, axis=0, keepdims=True)

        @pl.when(i == 0)
        def _():
            out_ref[...] = part

        @pl.when(i > 0)
        def _():
            out_ref[...] = out_ref[...] + part

    row_spec = pl.BlockSpec((B, _D), lambda i: (i, 0))
    col_spec = pl.BlockSpec((B, 1), lambda i: (i, 0))
    full = lambda r, cdim: pl.BlockSpec((r, cdim), lambda i: (0, 0))
    out = pl.pallas_call(
        body,
        grid=(_N // B,),
        in_specs=[row_spec, row_spec, col_spec, col_spec, row_spec, row_spec,
                  full(_D, _D), full(1, _D), full(_D, 4 * _D),
                  full(_D, 4 * _D), full(1, 4 * _D),
                  full(_D, _G), full(1, _G), full(_D, _G), full(1, _G)],
        out_specs=pl.BlockSpec((1, _G), lambda i: (0, 0)),
        out_shape=jax.ShapeDtypeStruct((1, _G), jnp.float32),
    )(s_parts[0], s_parts[1],
      deg_parts[0].reshape(_N, 1), deg_parts[1].reshape(_N, 1), h, c,
      w2_t, b2, wih_t, whh_t, bsum,
      gm_t, gm_b.reshape(1, _G), fm_t, fm_b.reshape(1, _G))
    return out.reshape(_G)


def _sc_edge(pi, pj, q, src, dst, z_nd, z_n):
    mesh = plsc.VectorSubcoreMesh(
        core_axis_name="c", subcore_axis_name="s",
        num_cores=_NC, num_subcores=_NS)

    @functools.partial(
        pl.kernel,
        mesh=mesh,
        out_type=(jax.ShapeDtypeStruct((_NC, _N, _D), jnp.float32),
                  jax.ShapeDtypeStruct((_NC, _N), jnp.float32)),
        scratch_types=[
            [pltpu.VMEM((_C,), jnp.int32) for _ in range(4)],
            [pltpu.VMEM((_C,), jnp.int32) for _ in range(4)],
            [pltpu.VMEM((_C, _D), jnp.float32) for _ in range(2)],
            [pltpu.VMEM((_C, _D), jnp.float32) for _ in range(2)],
            [pltpu.VMEM((_C, _D), jnp.float32) for _ in range(2)],
            [pltpu.VMEM((_C, _D), jnp.float32) for _ in range(2)],
            pltpu.VMEM((_C,), jnp.float32),
            pltpu.VMEM_SHARED((_N, _D), jnp.float32),
            pltpu.VMEM_SHARED((_N,), jnp.float32),
            [pltpu.SemaphoreType.DMA for _ in range(4)],
            [pltpu.SemaphoreType.DMA for _ in range(2)],
            [pltpu.SemaphoreType.DMA for _ in range(2)],
        ],
    )
    def k(pi_hbm, pj_hbm, q_hbm, src_hbm, dst_hbm, znd_hbm, zn_hbm,
          s_out, deg_out,
          idx_s, idx_d, buf_i, buf_j, buf_q, buf_r, ones_v, s_sh, deg_sh,
          sem_idx, sem_ld, sem_out):
        cid = lax.axis_index("c")
        sid = lax.axis_index("s")
        wid = cid * _NS + sid

        pltpu.sync_copy(znd_hbm.at[pl.ds(sid * _ROWS, _ROWS)],
                        s_sh.at[pl.ds(sid * _ROWS, _ROWS)])

        @pl.when(sid == _NS - 1)
        def _():
            pltpu.sync_copy(znd_hbm.at[pl.ds(_NS * _ROWS, _TAIL)],
                            s_sh.at[pl.ds(_NS * _ROWS, _TAIL)])

        @pl.when(sid == 0)
        def _():
            pltpu.sync_copy(zn_hbm, deg_sh)

        for t in range(_C // 16):
            ones_v[pl.ds(t * 16, 16)] = jnp.ones((16,), jnp.float32)
        if _C % 16:
            ones_v[pl.ds(_C - 16, 16)] = jnp.ones((16,), jnp.float32)

        plsc.subcore_barrier()

        base = wid * _EPT

        def issue_idx(ch, s):
            off = base + ch * _C
            pltpu.async_copy(src_hbm.at[pl.ds(off, _C)], idx_s[s], sem_idx[s])
            pltpu.async_copy(dst_hbm.at[pl.ds(off, _C)], idx_d[s], sem_idx[s])

        def wait_idx(s):
            pltpu.make_async_copy(src_hbm.at[pl.ds(0, _C)], idx_s[s], sem_idx[s]).wait()
            pltpu.make_async_copy(dst_hbm.at[pl.ds(0, _C)], idx_d[s], sem_idx[s]).wait()

        def issue_ld(ch, s, b):
            off = base + ch * _C
            pltpu.async_copy(pi_hbm.at[idx_d[s]], buf_i[b], sem_ld[b])
            pltpu.async_copy(pj_hbm.at[idx_s[s]], buf_j[b], sem_ld[b])
            pltpu.async_copy(q_hbm.at[pl.ds(off, _C)], buf_q[b], sem_ld[b])

        def wait_ld(b):
            pltpu.make_async_copy(pi_hbm.at[idx_d[0]], buf_i[b], sem_ld[b]).wait()
            pltpu.make_async_copy(pj_hbm.at[idx_s[0]], buf_j[b], sem_ld[b]).wait()
            pltpu.make_async_copy(q_hbm.at[pl.ds(0, _C)], buf_q[b], sem_ld[b]).wait()

        def issue_out(s, b):
            pltpu.async_copy(buf_r[b], s_sh.at[idx_d[s]], sem_out[b], add=True)
            pltpu.async_copy(ones_v, deg_sh.at[idx_d[s]], sem_out[b], add=True)

        def wait_out(b):
            pltpu.make_async_copy(buf_r[b], s_sh.at[idx_d[0]], sem_out[b]).wait()
            pltpu.make_async_copy(ones_v, deg_sh.at[idx_d[0]], sem_out[b]).wait()

        def compute(b):
            bi, bj, bq, br = buf_i[b], buf_j[b], buf_q[b], buf_r[b]

            def row(r, carry2):
                for t in range(_D // 16):
                    sl = pl.ds(t * 16, 16)
                    v = bi[r, sl] + bj[r, sl] + bq[r, sl]
                    br[r, sl] = jnp.maximum(v, 0.0)
                return carry2

            lax.fori_loop(0, _C, row, 0)

        issue_idx(0, 0)
        issue_idx(1, 1)
        wait_idx(0)
        issue_ld(0, 0, 0)
        wait_idx(1)
        issue_ld(1, 1, 1)

        def quad(i, carry):
            for kk in range(4):
                ch = 4 * i + kk
                b = kk % 2
                s = kk % 4
                s2 = (kk + 2) % 4

                @pl.when(ch < _NCHUNK)
                def _():
                    wait_ld(b)

                @pl.when(jnp.logical_and(ch >= 2, ch < _NCHUNK + 2))
                def _():
                    wait_out(b)

                @pl.when(ch + 2 < _NCHUNK)
                def _():
                    issue_idx(ch + 2, s2)

                @pl.when(ch < _NCHUNK)
                def _():
                    compute(b)
                    issue_out(s, b)

                @pl.when(ch + 2 < _NCHUNK)
                def _():
                    wait_idx(s2)
                    issue_ld(ch + 2, s2, b)

            return carry

        lax.fori_loop(0, (_NCHUNK + 3) // 4 + 1, quad, 0)

        plsc.subcore_barrier()

        pltpu.sync_copy(s_sh.at[pl.ds(sid * _ROWS, _ROWS)],
                        s_out.at[cid, pl.ds(sid * _ROWS, _ROWS)])

        @pl.when(sid == _NS - 1)
        def _():
            pltpu.sync_copy(s_sh.at[pl.ds(_NS * _ROWS, _TAIL)],
                            s_out.at[cid, pl.ds(_NS * _ROWS, _TAIL)])

        @pl.when(sid == 0)
        def _():
            pltpu.sync_copy(deg_sh, deg_out.at[cid])

    return k(pi, pj, q, src, dst, z_nd, z_n)


def kernel(x, edge_attr, edge_index, W1_0, b1_0, W2_0, b2_0, Wih_0, Whh_0,
           bih_0, bhh_0, W1_1, b1_1, W2_1, b2_1, Wih_1, Whh_1, bih_1, bhh_1,
           gm_W, gm_b, fm_W, fm_b):
    src = edge_index[0].astype(jnp.int32)
    dst = edge_index[1].astype(jnp.int32)
    z_nd = jnp.zeros((_N, _D), jnp.float32)
    z_n = jnp.zeros((_N,), jnp.float32)

    q0, q1 = _dense_q2(edge_attr,
                       W1_0[:, 2 * _D:].T, b1_0.reshape(1, _D),
                       W1_1[:, 2 * _D:].T, b1_1.reshape(1, _D))

    pi0, pj0 = _dense_pre(x, W1_0[:, :_D].T, W1_0[:, _D:2 * _D].T)
    s_parts0, deg_parts0 = _sc_edge(pi0, pj0, q0, src, dst, z_nd, z_n)
    h1, c1, pi1, pj1 = _post_pre(
        s_parts0, deg_parts0, x,
        W2_0.T, b2_0.reshape(1, _D), Wih_0.T, Whh_0.T,
        (bih_0 + bhh_0).reshape(1, 4 * _D),
        W1_1[:, :_D].T, W1_1[:, _D:2 * _D].T)
    s_parts1, deg_parts1 = _sc_edge(pi1, pj1, q1, src, dst, z_nd, z_n)
    return _post_readout(
        s_parts1, deg_parts1, h1, c1,
        W2_1.T, b2_1.reshape(1, _D), Wih_1.T, Whh_1.T,
        (bih_1 + bhh_1).reshape(1, 4 * _D),
        gm_W.T, gm_b, fm_W.T, fm_b)
